# Initial kernel scaffold; baseline (speedup 1.0000x reference)
import jax
import jax.numpy as jnp
from jax.experimental import pallas as pl


def kernel(x, edge_index, W1l, b1l, W1r, W2l, b2l, W2r, Wmu, bmu, Wlv, blv):
    raise NotImplementedError("write your pallas kernel here")


# speedup vs baseline: 4.1129x; 4.1129x over previous
import functools

import jax
import jax.numpy as jnp
from jax import lax
from jax.experimental import pallas as pl
from jax.experimental.pallas import tpu as pltpu
from jax.experimental.pallas import tpu_sc as plsc

_N = 10000
_E = 160000
_IN, _H1, _H2, _OUT = 256, 1024, 512, 256

_NC, _NS = 2, 16
_FC = 128
_B = 80
_EPT = _E // _NS
_NB = _EPT // _B
_NP = 10240
_RPT = _NP // _NS

_RB = 1000


def _sc_segsum(table_list, src, dst, zeros2d, zeros1d, with_count):
  C = len(table_list)
  mesh = plsc.VectorSubcoreMesh(
      core_axis_name="c", subcore_axis_name="s",
      num_cores=_NC, num_subcores=_NS)

  out_type = [jax.ShapeDtypeStruct((_NP, _FC), jnp.float32) for _ in range(C)]
  if with_count:
    out_type.append(jax.ShapeDtypeStruct((_NP,), jnp.float32))

  scratch = [
      pltpu.VMEM((_B,), jnp.int32),
      pltpu.VMEM((_B,), jnp.int32),
      pltpu.VMEM((_B, _FC), jnp.float32),
      pltpu.VMEM((_B,), jnp.float32),
      pltpu.VMEM_SHARED((_NP, _FC), jnp.float32),
  ]
  if with_count:
    scratch.append(pltpu.VMEM_SHARED((_NP,), jnp.float32))
  scratch.append(pltpu.SemaphoreType.DMA)

  def body(*refs):
    tables = refs[:C]
    src_hbm, dst_hbm, zeros2d_hbm = refs[C], refs[C + 1], refs[C + 2]
    i = C + 3
    if with_count:
      zeros1d_hbm = refs[i]
      i += 1
    outs = refs[i:i + C]
    i += C
    if with_count:
      cnt_hbm = refs[i]
      i += 1
    src_v, dst_v, rows_v, ones_v, acc = refs[i:i + 5]
    i += 5
    if with_count:
      cntacc = refs[i]
      i += 1
    sem = refs[i]

    cid = lax.axis_index("c")
    sid = lax.axis_index("s")
    ebase = sid * _EPT
    rbase = sid * _RPT

    if with_count:
      for l in range(_B // 16):
        ones_v[pl.ds(l * 16, 16)] = jnp.ones((16,), jnp.float32)

    for c in range(C):
      @pl.when(cid == (c % _NC))
      def _(c=c):
        pltpu.sync_copy(zeros2d_hbm, acc.at[pl.ds(rbase, _RPT)])
        if with_count and c == 0:
          @pl.when(sid == 0)
          def _():
            pltpu.sync_copy(zeros1d_hbm, cntacc)
        plsc.subcore_barrier()

        def batch(k, carry):
          off = ebase + k * _B
          pltpu.sync_copy(src_hbm.at[pl.ds(off, _B)], src_v)
          pltpu.sync_copy(dst_hbm.at[pl.ds(off, _B)], dst_v)
          pltpu.async_copy(tables[c].at[src_v], rows_v, sem).wait()
          pltpu.sync_copy(rows_v, acc.at[dst_v], add=True)
          if with_count and c == 0:
            pltpu.sync_copy(ones_v, cntacc.at[dst_v], add=True)
          return carry

        lax.fori_loop(0, _NB, batch, 0)
        plsc.subcore_barrier()
        pltpu.sync_copy(acc.at[pl.ds(rbase, _RPT)],
                        outs[c].at[pl.ds(rbase, _RPT)])
        if with_count and c == 0:
          @pl.when(sid == 0)
          def _():
            pltpu.sync_copy(cntacc, cnt_hbm)
        plsc.subcore_barrier()

    return None

  k = pl.kernel(body, out_type=tuple(out_type), mesh=mesh,
                scratch_types=tuple(scratch))
  args = list(table_list) + [src, dst, zeros2d]
  if with_count:
    args.append(zeros1d)
  return k(*args)


def _tc_layer1(x, agg0, agg1, cnt2, w1lT, b1l2, w1rT, w2lT, b2l2, w2rT):
  def body(x_ref, a0_ref, a1_ref, cnt_ref, w1l_ref, b1l_ref, w1r_ref,
           w2l_ref, b2l_ref, w2r_ref, hl0, hl1, hl2, hl3, hr_ref):
    rcp = 1.0 / jnp.maximum(cnt_ref[...], 1.0)
    mean = jnp.concatenate([a0_ref[...], a1_ref[...]], axis=1) * rcp
    t = (jnp.dot(mean, w1l_ref[...], preferred_element_type=jnp.float32)
         + jnp.dot(x_ref[...], w1r_ref[...], preferred_element_type=jnp.float32)
         + b1l_ref[...])
    h = jnp.maximum(t, 0.0)
    hl = jnp.dot(h, w2l_ref[...], preferred_element_type=jnp.float32)
    hr = (jnp.dot(h, w2r_ref[...], preferred_element_type=jnp.float32)
          + b2l_ref[...])
    hl0[...] = hl[:, 0:128]
    hl1[...] = hl[:, 128:256]
    hl2[...] = hl[:, 256:384]
    hl3[...] = hl[:, 384:512]
    hr_ref[...] = hr

  row = lambda i: (i, 0)
  fixed = lambda i: (0, 0)
  return pl.pallas_call(
      body,
      grid=(_N // _RB,),
      in_specs=[
          pl.BlockSpec((_RB, _IN), row),
          pl.BlockSpec((_RB, _FC), row),
          pl.BlockSpec((_RB, _FC), row),
          pl.BlockSpec((_RB, 1), row),
          pl.BlockSpec((_IN, _H1), fixed),
          pl.BlockSpec((1, _H1), fixed),
          pl.BlockSpec((_IN, _H1), fixed),
          pl.BlockSpec((_H1, _H2), fixed),
          pl.BlockSpec((1, _H2), fixed),
          pl.BlockSpec((_H1, _H2), fixed),
      ],
      out_specs=[
          pl.BlockSpec((_RB, _FC), row),
          pl.BlockSpec((_RB, _FC), row),
          pl.BlockSpec((_RB, _FC), row),
          pl.BlockSpec((_RB, _FC), row),
          pl.BlockSpec((_RB, _H2), row),
      ],
      out_shape=[
          jax.ShapeDtypeStruct((_N, _FC), jnp.float32),
          jax.ShapeDtypeStruct((_N, _FC), jnp.float32),
          jax.ShapeDtypeStruct((_N, _FC), jnp.float32),
          jax.ShapeDtypeStruct((_N, _FC), jnp.float32),
          jax.ShapeDtypeStruct((_N, _H2), jnp.float32),
      ],
  )(x, agg0, agg1, cnt2, w1lT, b1l2, w1rT, w2lT, b2l2, w2rT)


def _tc_layer2(a0, a1, a2, a3, cnt2, hr, wmuT, bmu2, wlvT, blv2):
  def body(a0_ref, a1_ref, a2_ref, a3_ref, cnt_ref, hr_ref,
           wmu_ref, bmu_ref, wlv_ref, blv_ref, mu_ref, lv_ref):
    rcp = 1.0 / jnp.maximum(cnt_ref[...], 1.0)
    agg = jnp.concatenate(
        [a0_ref[...], a1_ref[...], a2_ref[...], a3_ref[...]], axis=1)
    h2 = agg * rcp + hr_ref[...]
    mu_ref[...] = (jnp.dot(h2, wmu_ref[...],
                           preferred_element_type=jnp.float32) + bmu_ref[...])
    lv_ref[...] = (jnp.dot(h2, wlv_ref[...],
                           preferred_element_type=jnp.float32) + blv_ref[...])

  row = lambda i: (i, 0)
  fixed = lambda i: (0, 0)
  return pl.pallas_call(
      body,
      grid=(_N // _RB,),
      in_specs=[
          pl.BlockSpec((_RB, _FC), row),
          pl.BlockSpec((_RB, _FC), row),
          pl.BlockSpec((_RB, _FC), row),
          pl.BlockSpec((_RB, _FC), row),
          pl.BlockSpec((_RB, 1), row),
          pl.BlockSpec((_RB, _H2), row),
          pl.BlockSpec((_H2, _OUT), fixed),
          pl.BlockSpec((1, _OUT), fixed),
          pl.BlockSpec((_H2, _OUT), fixed),
          pl.BlockSpec((1, _OUT), fixed),
      ],
      out_specs=[
          pl.BlockSpec((_RB, _OUT), row),
          pl.BlockSpec((_RB, _OUT), row),
      ],
      out_shape=[
          jax.ShapeDtypeStruct((_N, _OUT), jnp.float32),
          jax.ShapeDtypeStruct((_N, _OUT), jnp.float32),
      ],
  )(a0, a1, a2, a3, cnt2, hr, wmuT, bmu2, wlvT, blv2)


def kernel(x, edge_index, W1l, b1l, W1r, W2l, b2l, W2r, Wmu, bmu, Wlv, blv):
  f32 = jnp.float32
  src = edge_index[0]
  dst = edge_index[1]
  x0 = x[:, :_FC]
  x1 = x[:, _FC:]
  zeros2d = jnp.zeros((_RPT, _FC), f32)
  zeros1d = jnp.zeros((_NP,), f32)

  agg10, agg11, cnt = _sc_segsum([x0, x1], src, dst, zeros2d, zeros1d, True)
  cnt2 = cnt.reshape(_NP, 1)

  hl0, hl1, hl2, hl3, hr = _tc_layer1(
      x, agg10, agg11, cnt2, W1l.T, b1l.reshape(1, -1), W1r.T,
      W2l.T, b2l.reshape(1, -1), W2r.T)

  a20, a21, a22, a23 = _sc_segsum(
      [hl0, hl1, hl2, hl3], src, dst, zeros2d, zeros1d, False)

  mu, lv = _tc_layer2(
      a20, a21, a22, a23, cnt2, hr, Wmu.T, bmu.reshape(1, -1),
      Wlv.T, blv.reshape(1, -1))
  return (mu, lv)


# speedup vs baseline: 7.7329x; 1.8801x over previous
import functools

import jax
import jax.numpy as jnp
from jax import lax
from jax.experimental import pallas as pl
from jax.experimental.pallas import tpu as pltpu
from jax.experimental.pallas import tpu_sc as plsc

_N = 10000
_E = 160000
_IN, _H1, _H2, _OUT = 256, 1024, 512, 256

_NC, _NS = 2, 16
_FC = 128
_B = 128
_NBT = _E // _B
_NP = 10240
_RPT = _NP // _NS

_RB = 1000


def _sc_segsum(table_list, src, dst, zeros2d, zeros1d, with_count):
  C = len(table_list)
  mesh = plsc.VectorSubcoreMesh(
      core_axis_name="c", subcore_axis_name="s",
      num_cores=_NC, num_subcores=_NS)

  out_type = [jax.ShapeDtypeStruct((_NP, _FC), jnp.float32) for _ in range(C)]
  if with_count:
    out_type.append(jax.ShapeDtypeStruct((_NP,), jnp.float32))

  scratch = [
      pltpu.VMEM((_B,), jnp.int32),
      pltpu.VMEM((_B,), jnp.int32),
      pltpu.VMEM((_B,), jnp.int32),
      pltpu.VMEM((_B,), jnp.int32),
      pltpu.VMEM((_B, _FC), jnp.float32),
      pltpu.VMEM((_B, _FC), jnp.float32),
      pltpu.VMEM((_B,), jnp.float32),
      pltpu.VMEM_SHARED((_NP, _FC), jnp.float32),
  ]
  if with_count:
    scratch.append(pltpu.VMEM_SHARED((_NP,), jnp.float32))
  scratch.append(pltpu.SemaphoreType.DMA)
  scratch.append(pltpu.SemaphoreType.DMA)

  def body(*refs):
    tables = refs[:C]
    src_hbm, dst_hbm, zeros2d_hbm = refs[C], refs[C + 1], refs[C + 2]
    i = C + 3
    if with_count:
      zeros1d_hbm = refs[i]
      i += 1
    outs = refs[i:i + C]
    i += C
    if with_count:
      cnt_hbm = refs[i]
      i += 1
    src_v = (refs[i], refs[i + 2])
    dst_v = (refs[i + 1], refs[i + 3])
    rows_v = (refs[i + 4], refs[i + 5])
    ones_v = refs[i + 6]
    acc = refs[i + 7]
    i += 8
    if with_count:
      cntacc = refs[i]
      i += 1
    sem = (refs[i], refs[i + 1])

    cid = lax.axis_index("c")
    sid = lax.axis_index("s")
    rbase = sid * _RPT
    nb = 78 + (sid < _NBT % _NS).astype(jnp.int32)

    if with_count:
      for l in range(_B // 16):
        ones_v[pl.ds(l * 16, 16)] = jnp.ones((16,), jnp.float32)

    for c in range(C):
      @pl.when(cid == (c % _NC))
      def _(c=c):
        pltpu.sync_copy(zeros2d_hbm, acc.at[pl.ds(rbase, _RPT)])
        if with_count and c == 0:
          @pl.when(sid == 0)
          def _():
            pltpu.sync_copy(zeros1d_hbm, cntacc)
        plsc.subcore_barrier()

        def fetch(m, p):
          off = (sid + m * _NS) * _B
          pltpu.sync_copy(src_hbm.at[pl.ds(off, _B)], src_v[p])
          pltpu.sync_copy(dst_hbm.at[pl.ds(off, _B)], dst_v[p])
          pltpu.async_copy(tables[c].at[src_v[p]], rows_v[p], sem[p])

        def drain_scatter(p):
          pltpu.make_async_copy(tables[c].at[src_v[p]], rows_v[p],
                                sem[p]).wait()
          pltpu.sync_copy(rows_v[p], acc.at[dst_v[p]], add=True)
          if with_count and c == 0:
            pltpu.sync_copy(ones_v, cntacc.at[dst_v[p]], add=True)

        fetch(0, 0)

        def pair(k2, carry):
          for p in range(2):
            m = 2 * k2 + p

            @pl.when(m < nb)
            def _(m=m, p=p):
              @pl.when(m + 1 < nb)
              def _(m=m, p=p):
                fetch(m + 1, 1 - p)
              drain_scatter(p)
          return carry

        lax.fori_loop(0, (nb + 1) // 2, pair, 0)
        plsc.subcore_barrier()
        pltpu.sync_copy(acc.at[pl.ds(rbase, _RPT)],
                        outs[c].at[pl.ds(rbase, _RPT)])
        if with_count and c == 0:
          @pl.when(sid == 0)
          def _():
            pltpu.sync_copy(cntacc, cnt_hbm)
        plsc.subcore_barrier()

    return None

  k = pl.kernel(body, out_type=tuple(out_type), mesh=mesh,
                scratch_types=tuple(scratch))
  args = list(table_list) + [src, dst, zeros2d]
  if with_count:
    args.append(zeros1d)
  return k(*args)


def _tc_layer1(x, agg0, agg1, cnt2, w1lT, b1l2, w1rT, w2lT, b2l2, w2rT):
  def body(x_ref, a0_ref, a1_ref, cnt_ref, w1l_ref, b1l_ref, w1r_ref,
           w2l_ref, b2l_ref, w2r_ref, hl0, hl1, hl2, hl3, hr_ref):
    rcp = 1.0 / jnp.maximum(cnt_ref[...], 1.0)
    mean = jnp.concatenate([a0_ref[...], a1_ref[...]], axis=1) * rcp
    t = (jnp.dot(mean, w1l_ref[...], preferred_element_type=jnp.float32)
         + jnp.dot(x_ref[...], w1r_ref[...], preferred_element_type=jnp.float32)
         + b1l_ref[...])
    h = jnp.maximum(t, 0.0)
    hl = jnp.dot(h, w2l_ref[...], preferred_element_type=jnp.float32)
    hr = (jnp.dot(h, w2r_ref[...], preferred_element_type=jnp.float32)
          + b2l_ref[...])
    hl0[...] = hl[:, 0:128]
    hl1[...] = hl[:, 128:256]
    hl2[...] = hl[:, 256:384]
    hl3[...] = hl[:, 384:512]
    hr_ref[...] = hr

  row = lambda i: (i, 0)
  fixed = lambda i: (0, 0)
  return pl.pallas_call(
      body,
      grid=(_N // _RB,),
      in_specs=[
          pl.BlockSpec((_RB, _IN), row),
          pl.BlockSpec((_RB, _FC), row),
          pl.BlockSpec((_RB, _FC), row),
          pl.BlockSpec((_RB, 1), row),
          pl.BlockSpec((_IN, _H1), fixed),
          pl.BlockSpec((1, _H1), fixed),
          pl.BlockSpec((_IN, _H1), fixed),
          pl.BlockSpec((_H1, _H2), fixed),
          pl.BlockSpec((1, _H2), fixed),
          pl.BlockSpec((_H1, _H2), fixed),
      ],
      out_specs=[
          pl.BlockSpec((_RB, _FC), row),
          pl.BlockSpec((_RB, _FC), row),
          pl.BlockSpec((_RB, _FC), row),
          pl.BlockSpec((_RB, _FC), row),
          pl.BlockSpec((_RB, _H2), row),
      ],
      out_shape=[
          jax.ShapeDtypeStruct((_N, _FC), jnp.float32),
          jax.ShapeDtypeStruct((_N, _FC), jnp.float32),
          jax.ShapeDtypeStruct((_N, _FC), jnp.float32),
          jax.ShapeDtypeStruct((_N, _FC), jnp.float32),
          jax.ShapeDtypeStruct((_N, _H2), jnp.float32),
      ],
  )(x, agg0, agg1, cnt2, w1lT, b1l2, w1rT, w2lT, b2l2, w2rT)


def _tc_layer2(a0, a1, a2, a3, cnt2, hr, wmuT, bmu2, wlvT, blv2):
  def body(a0_ref, a1_ref, a2_ref, a3_ref, cnt_ref, hr_ref,
           wmu_ref, bmu_ref, wlv_ref, blv_ref, mu_ref, lv_ref):
    rcp = 1.0 / jnp.maximum(cnt_ref[...], 1.0)
    agg = jnp.concatenate(
        [a0_ref[...], a1_ref[...], a2_ref[...], a3_ref[...]], axis=1)
    h2 = agg * rcp + hr_ref[...]
    mu_ref[...] = (jnp.dot(h2, wmu_ref[...],
                           preferred_element_type=jnp.float32) + bmu_ref[...])
    lv_ref[...] = (jnp.dot(h2, wlv_ref[...],
                           preferred_element_type=jnp.float32) + blv_ref[...])

  row = lambda i: (i, 0)
  fixed = lambda i: (0, 0)
  return pl.pallas_call(
      body,
      grid=(_N // _RB,),
      in_specs=[
          pl.BlockSpec((_RB, _FC), row),
          pl.BlockSpec((_RB, _FC), row),
          pl.BlockSpec((_RB, _FC), row),
          pl.BlockSpec((_RB, _FC), row),
          pl.BlockSpec((_RB, 1), row),
          pl.BlockSpec((_RB, _H2), row),
          pl.BlockSpec((_H2, _OUT), fixed),
          pl.BlockSpec((1, _OUT), fixed),
          pl.BlockSpec((_H2, _OUT), fixed),
          pl.BlockSpec((1, _OUT), fixed),
      ],
      out_specs=[
          pl.BlockSpec((_RB, _OUT), row),
          pl.BlockSpec((_RB, _OUT), row),
      ],
      out_shape=[
          jax.ShapeDtypeStruct((_N, _OUT), jnp.float32),
          jax.ShapeDtypeStruct((_N, _OUT), jnp.float32),
      ],
  )(a0, a1, a2, a3, cnt2, hr, wmuT, bmu2, wlvT, blv2)


def kernel(x, edge_index, W1l, b1l, W1r, W2l, b2l, W2r, Wmu, bmu, Wlv, blv):
  f32 = jnp.float32
  src = edge_index[0]
  dst = edge_index[1]
  x0 = x[:, :_FC]
  x1 = x[:, _FC:]
  zeros2d = jnp.zeros((_RPT, _FC), f32)
  zeros1d = jnp.zeros((_NP,), f32)

  agg10, agg11, cnt = _sc_segsum([x0, x1], src, dst, zeros2d, zeros1d, True)
  cnt2 = cnt.reshape(_NP, 1)

  hl0, hl1, hl2, hl3, hr = _tc_layer1(
      x, agg10, agg11, cnt2, W1l.T, b1l.reshape(1, -1), W1r.T,
      W2l.T, b2l.reshape(1, -1), W2r.T)

  a20, a21, a22, a23 = _sc_segsum(
      [hl0, hl1, hl2, hl3], src, dst, zeros2d, zeros1d, False)

  mu, lv = _tc_layer2(
      a20, a21, a22, a23, cnt2, hr, Wmu.T, bmu.reshape(1, -1),
      Wlv.T, blv.reshape(1, -1))
  return (mu, lv)


# speedup vs baseline: 7.7644x; 1.0041x over previous
import functools

import jax
import jax.numpy as jnp
from jax import lax
from jax.experimental import pallas as pl
from jax.experimental.pallas import tpu as pltpu
from jax.experimental.pallas import tpu_sc as plsc

_N = 10000
_E = 160000
_IN, _H1, _H2, _OUT = 256, 1024, 512, 256

_NC, _NS = 2, 16
_FC = 128
_B = 128
_NBT = _E // _B
_NP = 10240
_RPT = _NP // _NS

_RB = 1000


def _sc_segsum(table_list, src, dst, zeros2d, zeros1d, with_count):
  C = len(table_list)
  mesh = plsc.VectorSubcoreMesh(
      core_axis_name="c", subcore_axis_name="s",
      num_cores=_NC, num_subcores=_NS)

  out_type = [jax.ShapeDtypeStruct((_NP, _FC), jnp.float32) for _ in range(C)]
  if with_count:
    out_type.append(jax.ShapeDtypeStruct((_NP,), jnp.float32))

  scratch = [
      pltpu.VMEM((_B,), jnp.int32),
      pltpu.VMEM((_B,), jnp.int32),
      pltpu.VMEM((_B,), jnp.int32),
      pltpu.VMEM((_B,), jnp.int32),
      pltpu.VMEM((_B, _FC), jnp.float32),
      pltpu.VMEM((_B, _FC), jnp.float32),
      pltpu.VMEM((_B,), jnp.float32),
      pltpu.VMEM_SHARED((_NP, _FC), jnp.float32),
  ]
  if with_count:
    scratch.append(pltpu.VMEM_SHARED((_NP,), jnp.float32))
  scratch.extend([pltpu.SemaphoreType.DMA] * 4)

  def body(*refs):
    tables = refs[:C]
    src_hbm, dst_hbm, zeros2d_hbm = refs[C], refs[C + 1], refs[C + 2]
    i = C + 3
    if with_count:
      zeros1d_hbm = refs[i]
      i += 1
    outs = refs[i:i + C]
    i += C
    if with_count:
      cnt_hbm = refs[i]
      i += 1
    src_v = (refs[i], refs[i + 2])
    dst_v = (refs[i + 1], refs[i + 3])
    rows_v = (refs[i + 4], refs[i + 5])
    ones_v = refs[i + 6]
    acc = refs[i + 7]
    i += 8
    if with_count:
      cntacc = refs[i]
      i += 1
    semg = (refs[i], refs[i + 1])
    sems = (refs[i + 2], refs[i + 3])

    cid = lax.axis_index("c")
    sid = lax.axis_index("s")
    rbase = sid * _RPT
    nb = 78 + (sid < _NBT % _NS).astype(jnp.int32)

    if with_count:
      for l in range(_B // 16):
        ones_v[pl.ds(l * 16, 16)] = jnp.ones((16,), jnp.float32)

    for c in range(C):
      @pl.when(cid == (c % _NC))
      def _(c=c):
        pltpu.sync_copy(zeros2d_hbm, acc.at[pl.ds(rbase, _RPT)])
        if with_count and c == 0:
          @pl.when(sid == 0)
          def _():
            pltpu.sync_copy(zeros1d_hbm, cntacc)
        plsc.subcore_barrier()

        def fetch(m, p):
          off = (sid + m * _NS) * _B
          pltpu.sync_copy(src_hbm.at[pl.ds(off, _B)], src_v[p])
          pltpu.sync_copy(dst_hbm.at[pl.ds(off, _B)], dst_v[p])
          pltpu.async_copy(tables[c].at[src_v[p]], rows_v[p], semg[p])

        def wait_scatter(p):
          pltpu.make_async_copy(rows_v[p], acc.at[dst_v[p]], sems[p]).wait()
          if with_count and c == 0:
            pltpu.make_async_copy(ones_v, cntacc.at[dst_v[p]],
                                  sems[p]).wait()

        def issue_scatter(p):
          pltpu.async_copy(rows_v[p], acc.at[dst_v[p]], sems[p], add=True)
          if with_count and c == 0:
            pltpu.async_copy(ones_v, cntacc.at[dst_v[p]], sems[p], add=True)

        fetch(0, 0)

        def pair(k2, carry):
          for p in range(2):
            m = 2 * k2 + p

            @pl.when(m < nb)
            def _(m=m, p=p):
              @pl.when(m + 1 < nb)
              def _(m=m, p=p):
                @pl.when(m >= 1)
                def _(p=p):
                  wait_scatter(1 - p)
                fetch(m + 1, 1 - p)
              pltpu.make_async_copy(tables[c].at[src_v[p]], rows_v[p],
                                    semg[p]).wait()
              issue_scatter(p)
          return carry

        lax.fori_loop(0, (nb + 1) // 2, pair, 0)
        wait_scatter(0)
        wait_scatter(1)
        plsc.subcore_barrier()
        pltpu.sync_copy(acc.at[pl.ds(rbase, _RPT)],
                        outs[c].at[pl.ds(rbase, _RPT)])
        if with_count and c == 0:
          @pl.when(sid == 0)
          def _():
            pltpu.sync_copy(cntacc, cnt_hbm)
        plsc.subcore_barrier()

    return None

  k = pl.kernel(body, out_type=tuple(out_type), mesh=mesh,
                scratch_types=tuple(scratch))
  args = list(table_list) + [src, dst, zeros2d]
  if with_count:
    args.append(zeros1d)
  return k(*args)


def _tc_layer1(x, agg0, agg1, cnt2, w1lT, b1l2, w1rT, w2lT, b2l2, w2rT):
  def body(x_ref, a0_ref, a1_ref, cnt_ref, w1l_ref, b1l_ref, w1r_ref,
           w2l_ref, b2l_ref, w2r_ref, hl0, hl1, hl2, hl3, hr_ref):
    rcp = 1.0 / jnp.maximum(cnt_ref[...], 1.0)
    mean = jnp.concatenate([a0_ref[...], a1_ref[...]], axis=1) * rcp
    t = (jnp.dot(mean, w1l_ref[...], preferred_element_type=jnp.float32)
         + jnp.dot(x_ref[...], w1r_ref[...], preferred_element_type=jnp.float32)
         + b1l_ref[...])
    h = jnp.maximum(t, 0.0)
    hl = jnp.dot(h, w2l_ref[...], preferred_element_type=jnp.float32)
    hr = (jnp.dot(h, w2r_ref[...], preferred_element_type=jnp.float32)
          + b2l_ref[...])
    hl0[...] = hl[:, 0:128]
    hl1[...] = hl[:, 128:256]
    hl2[...] = hl[:, 256:384]
    hl3[...] = hl[:, 384:512]
    hr_ref[...] = hr

  row = lambda i: (i, 0)
  fixed = lambda i: (0, 0)
  return pl.pallas_call(
      body,
      grid=(_N // _RB,),
      in_specs=[
          pl.BlockSpec((_RB, _IN), row),
          pl.BlockSpec((_RB, _FC), row),
          pl.BlockSpec((_RB, _FC), row),
          pl.BlockSpec((_RB, 1), row),
          pl.BlockSpec((_IN, _H1), fixed),
          pl.BlockSpec((1, _H1), fixed),
          pl.BlockSpec((_IN, _H1), fixed),
          pl.BlockSpec((_H1, _H2), fixed),
          pl.BlockSpec((1, _H2), fixed),
          pl.BlockSpec((_H1, _H2), fixed),
      ],
      out_specs=[
          pl.BlockSpec((_RB, _FC), row),
          pl.BlockSpec((_RB, _FC), row),
          pl.BlockSpec((_RB, _FC), row),
          pl.BlockSpec((_RB, _FC), row),
          pl.BlockSpec((_RB, _H2), row),
      ],
      out_shape=[
          jax.ShapeDtypeStruct((_N, _FC), jnp.float32),
          jax.ShapeDtypeStruct((_N, _FC), jnp.float32),
          jax.ShapeDtypeStruct((_N, _FC), jnp.float32),
          jax.ShapeDtypeStruct((_N, _FC), jnp.float32),
          jax.ShapeDtypeStruct((_N, _H2), jnp.float32),
      ],
  )(x, agg0, agg1, cnt2, w1lT, b1l2, w1rT, w2lT, b2l2, w2rT)


def _tc_layer2(a0, a1, a2, a3, cnt2, hr, wmuT, bmu2, wlvT, blv2):
  def body(a0_ref, a1_ref, a2_ref, a3_ref, cnt_ref, hr_ref,
           wmu_ref, bmu_ref, wlv_ref, blv_ref, mu_ref, lv_ref):
    rcp = 1.0 / jnp.maximum(cnt_ref[...], 1.0)
    agg = jnp.concatenate(
        [a0_ref[...], a1_ref[...], a2_ref[...], a3_ref[...]], axis=1)
    h2 = agg * rcp + hr_ref[...]
    mu_ref[...] = (jnp.dot(h2, wmu_ref[...],
                           preferred_element_type=jnp.float32) + bmu_ref[...])
    lv_ref[...] = (jnp.dot(h2, wlv_ref[...],
                           preferred_element_type=jnp.float32) + blv_ref[...])

  row = lambda i: (i, 0)
  fixed = lambda i: (0, 0)
  return pl.pallas_call(
      body,
      grid=(_N // _RB,),
      in_specs=[
          pl.BlockSpec((_RB, _FC), row),
          pl.BlockSpec((_RB, _FC), row),
          pl.BlockSpec((_RB, _FC), row),
          pl.BlockSpec((_RB, _FC), row),
          pl.BlockSpec((_RB, 1), row),
          pl.BlockSpec((_RB, _H2), row),
          pl.BlockSpec((_H2, _OUT), fixed),
          pl.BlockSpec((1, _OUT), fixed),
          pl.BlockSpec((_H2, _OUT), fixed),
          pl.BlockSpec((1, _OUT), fixed),
      ],
      out_specs=[
          pl.BlockSpec((_RB, _OUT), row),
          pl.BlockSpec((_RB, _OUT), row),
      ],
      out_shape=[
          jax.ShapeDtypeStruct((_N, _OUT), jnp.float32),
          jax.ShapeDtypeStruct((_N, _OUT), jnp.float32),
      ],
  )(a0, a1, a2, a3, cnt2, hr, wmuT, bmu2, wlvT, blv2)


def kernel(x, edge_index, W1l, b1l, W1r, W2l, b2l, W2r, Wmu, bmu, Wlv, blv):
  f32 = jnp.float32
  src = edge_index[0]
  dst = edge_index[1]
  x0 = x[:, :_FC]
  x1 = x[:, _FC:]
  zeros2d = jnp.zeros((_RPT, _FC), f32)
  zeros1d = jnp.zeros((_NP,), f32)

  agg10, agg11, cnt = _sc_segsum([x0, x1], src, dst, zeros2d, zeros1d, True)
  cnt2 = cnt.reshape(_NP, 1)

  hl0, hl1, hl2, hl3, hr = _tc_layer1(
      x, agg10, agg11, cnt2, W1l.T, b1l.reshape(1, -1), W1r.T,
      W2l.T, b2l.reshape(1, -1), W2r.T)

  a20, a21, a22, a23 = _sc_segsum(
      [hl0, hl1, hl2, hl3], src, dst, zeros2d, zeros1d, False)

  mu, lv = _tc_layer2(
      a20, a21, a22, a23, cnt2, hr, Wmu.T, bmu.reshape(1, -1),
      Wlv.T, blv.reshape(1, -1))
  return (mu, lv)
